# baseline (speedup 1.0000x reference)
import math

import jax
import jax.numpy as jnp
from jax.experimental import pallas as pl
from jax.experimental.pallas import tpu as pltpu

B, L, S, H, D = 2, 8192, 128, 16, 64
TOPK = 16
BLOCK_SIZE = 64
WINDOW = 64
SM_SCALE = 1.0 / math.sqrt(D)

LT = 1024
NC = LT // 128

PACKED_NINF = -2147483648


def _batcher_pairs(n):
    pairs = []
    p = 1
    while p < n:
        k = p
        while k >= 1:
            for j in range(k % p, n - k, 2 * k):
                for i in range(0, min(k, n - j - k)):
                    if (i + j) // (2 * p) == (i + j + k) // (2 * p):
                        pairs.append((i + j, i + j + k))
            k //= 2
        p *= 2
    return pairs


_PAIRS16 = _batcher_pairs(TOPK)


def _ce(p, a, b):
    hi = jnp.maximum(p[a], p[b])
    lo = jnp.minimum(p[a], p[b])
    p[a], p[b] = hi, lo


def _merge_top16(ap, bp):
    c = [jnp.maximum(ap[t], bp[TOPK - 1 - t]) for t in range(TOPK)]
    for d in (8, 4, 2, 1):
        for t in range(TOPK):
            if t & d == 0:
                _ce(c, t, t + d)
    return c


def _topk_kernel(q_ref, k_ref, ov_ref, oi_ref, sc_ref):
    l_idx = pl.program_id(1)
    h = pl.program_id(2)
    neg_inf = jnp.float32(-jnp.inf)

    q_h = q_ref[0, :, h, :]
    k_h = k_ref[0, :, h, :]
    res = jax.lax.dot_general(
        k_h, q_h,
        dimension_numbers=(((1,), (1,)), ((), ())),
        preferred_element_type=jnp.float32,
    )

    u = jax.lax.bitcast_convert_type(res, jnp.int32)
    t = u ^ ((u >> 31) & jnp.int32(0x7FFFFFFF))
    ts = jax.lax.broadcasted_iota(jnp.int32, (S, LT), 0)
    packed = (t & jnp.int32(~127)) | (jnp.int32(127) - ts)
    tq = l_idx * LT + jax.lax.broadcasted_iota(jnp.int32, (S, LT), 1)
    packed = jnp.where(BLOCK_SIZE * ts + (BLOCK_SIZE + WINDOW - 1) <= tq,
                       packed, jnp.int32(PACKED_NINF))
    for j in range(NC):
        sc_ref[:, j, :] = packed[:, j * 128:(j + 1) * 128]

    def sorted_group(g):
        p = [sc_ref[g * TOPK + t] for t in range(TOPK)]
        for a, b in _PAIRS16:
            _ce(p, a, b)
        return p

    p0 = sorted_group(0)

    def body(g, carry):
        return tuple(_merge_top16(list(carry), sorted_group(g)))

    fp = list(jax.lax.fori_loop(1, l_idx + 1, body, tuple(p0)))

    for t in range(TOPK):
        pk = fp[t]
        is_ninf = pk == jnp.int32(PACKED_NINF)
        tv = pk & jnp.int32(~127)
        uv = tv ^ ((tv >> 31) & jnp.int32(0x7FFFFFFF))
        vt = jax.lax.bitcast_convert_type(uv, jnp.float32)
        vt = jnp.where(is_ninf, neg_inf, vt)
        it = jnp.where(is_ninf, jnp.int32(-1),
                       jnp.int32(127) - (pk & jnp.int32(127)))
        ov_ref[0, 0, h * TOPK + t] = vt
        oi_ref[0, 0, h * TOPK + t] = it


@jax.jit
def kernel(q, k):
    k_scaled = k * jnp.float32(SM_SCALE)
    nl = L // LT
    grid = (B, nl, H)
    out_shape = (
        jax.ShapeDtypeStruct((B, nl, H * TOPK, NC, 128), jnp.float32),
        jax.ShapeDtypeStruct((B, nl, H * TOPK, NC, 128), jnp.int32),
    )
    q_spec = pl.BlockSpec((1, LT, H, D), lambda b, l, h: (b, l, 0, 0))
    k_spec = pl.BlockSpec((1, S, H, D), lambda b, l, h: (b, 0, 0, 0))
    o_spec = pl.BlockSpec((1, 1, H * TOPK, NC, 128), lambda b, l, h: (b, l, 0, 0, 0))
    ov, oi = pl.pallas_call(
        _topk_kernel,
        grid=grid,
        in_specs=[q_spec, k_spec],
        out_specs=(o_spec, o_spec),
        out_shape=out_shape,
        scratch_shapes=[pltpu.VMEM((S, NC, 128), jnp.int32)],
    )(q, k_scaled)
    ov = ov.transpose(0, 1, 3, 4, 2).reshape(B, L, H, TOPK)
    oi = oi.transpose(0, 1, 3, 4, 2).reshape(B, L, H, TOPK)
    return ov, oi


# speedup vs baseline: 1.0979x; 1.0979x over previous
import math

import jax
import jax.numpy as jnp
from jax.experimental import pallas as pl
from jax.experimental.pallas import tpu as pltpu

B, L, S, H, D = 2, 8192, 128, 16, 64
TOPK = 16
BLOCK_SIZE = 64
WINDOW = 64
SM_SCALE = 1.0 / math.sqrt(D)

LT = 1024
NC = LT // 128


def _batcher_pairs(n):
    pairs = []
    p = 1
    while p < n:
        k = p
        while k >= 1:
            for j in range(k % p, n - k, 2 * k):
                for i in range(0, min(k, n - j - k)):
                    if (i + j) // (2 * p) == (i + j + k) // (2 * p):
                        pairs.append((i + j, i + j + k))
            k //= 2
        p *= 2
    return pairs


_PAIRS16 = _batcher_pairs(TOPK)


def _ce(p, a, b):
    hi = jnp.maximum(p[a], p[b])
    lo = jnp.minimum(p[a], p[b])
    p[a], p[b] = hi, lo


def _merge_top16(ap, bp):
    c = [jnp.maximum(ap[t], bp[TOPK - 1 - t]) for t in range(TOPK)]
    for d in (8, 4, 2, 1):
        for t in range(TOPK):
            if t & d == 0:
                _ce(c, t, t + d)
    return c


def _topk_kernel(q_ref, k_ref, ov_ref, oi_ref, sc_ref):
    l_idx = pl.program_id(1)
    h = pl.program_id(2)
    neg_inf = jnp.float32(-jnp.inf)

    q_h = q_ref[0, :, h, :]
    k_h = k_ref[0, :, h, :]
    res = jax.lax.dot_general(
        k_h, q_h,
        dimension_numbers=(((1,), (1,)), ((), ())),
        preferred_element_type=jnp.float32,
    )
    for j in range(NC):
        sc_ref[:, j, :] = res[:, j * 128:(j + 1) * 128]

    tq = (l_idx * LT
          + 128 * jax.lax.broadcasted_iota(jnp.int32, (NC, 128), 0)
          + jax.lax.broadcasted_iota(jnp.int32, (NC, 128), 1))

    def sorted_group(g):
        p = []
        for t in range(TOPK):
            s = g * TOPK + t
            raw = jax.lax.bitcast_convert_type(sc_ref[s], jnp.int32)
            pk = jax.lax.bitcast_convert_type(
                (raw & jnp.int32(~127)) | (jnp.int32(127) - s), jnp.float32)
            p.append(jnp.where(tq >= BLOCK_SIZE * s + (BLOCK_SIZE + WINDOW - 1),
                               pk, neg_inf))
        for a, b in _PAIRS16:
            _ce(p, a, b)
        return p

    p0 = sorted_group(0)

    def body(g, carry):
        return tuple(_merge_top16(list(carry), sorted_group(g)))

    fp = list(jax.lax.fori_loop(1, l_idx + 1, body, tuple(p0)))

    for t in range(TOPK):
        pk = fp[t]
        is_ninf = pk == neg_inf
        bits = jax.lax.bitcast_convert_type(pk, jnp.int32)
        vt = jax.lax.bitcast_convert_type(bits & jnp.int32(~127), jnp.float32)
        vt = jnp.where(is_ninf, neg_inf, vt)
        it = jnp.where(is_ninf, jnp.int32(-1),
                       jnp.int32(127) - (bits & jnp.int32(127)))
        ov_ref[0, 0, h * TOPK + t] = vt
        oi_ref[0, 0, h * TOPK + t] = it


@jax.jit
def kernel(q, k):
    k_scaled = k * jnp.float32(SM_SCALE)
    nl = L // LT
    grid = (B, nl, H)
    out_shape = (
        jax.ShapeDtypeStruct((B, nl, H * TOPK, NC, 128), jnp.float32),
        jax.ShapeDtypeStruct((B, nl, H * TOPK, NC, 128), jnp.int32),
    )
    q_spec = pl.BlockSpec((1, LT, H, D), lambda b, l, h: (b, l, 0, 0))
    k_spec = pl.BlockSpec((1, S, H, D), lambda b, l, h: (b, 0, 0, 0))
    o_spec = pl.BlockSpec((1, 1, H * TOPK, NC, 128), lambda b, l, h: (b, l, 0, 0, 0))
    ov, oi = pl.pallas_call(
        _topk_kernel,
        grid=grid,
        in_specs=[q_spec, k_spec],
        out_specs=(o_spec, o_spec),
        out_shape=out_shape,
        scratch_shapes=[pltpu.VMEM((S, NC, 128), jnp.float32)],
    )(q, k_scaled)
    ov = ov.transpose(0, 1, 3, 4, 2).reshape(B, L, H, TOPK)
    oi = oi.transpose(0, 1, 3, 4, 2).reshape(B, L, H, TOPK)
    return ov, oi


# speedup vs baseline: 1.7288x; 1.5746x over previous
import math

import jax
import jax.numpy as jnp
from jax.experimental import pallas as pl
from jax.experimental.pallas import tpu as pltpu

B, L, S, H, D = 2, 8192, 128, 16, 64
TOPK = 16
BLOCK_SIZE = 64
WINDOW = 64
SM_SCALE = 1.0 / math.sqrt(D)

LT = 1024
NC = LT // 128


def _batcher_pairs(n):
    pairs = []
    p = 1
    while p < n:
        k = p
        while k >= 1:
            for j in range(k % p, n - k, 2 * k):
                for i in range(0, min(k, n - j - k)):
                    if (i + j) // (2 * p) == (i + j + k) // (2 * p):
                        pairs.append((i + j, i + j + k))
            k //= 2
        p *= 2
    return pairs


_PAIRS16 = _batcher_pairs(TOPK)


def _ce(p, a, b):
    hi = jnp.maximum(p[a], p[b])
    lo = jnp.minimum(p[a], p[b])
    p[a], p[b] = hi, lo


def _merge_top16(ap, bp):
    c = [jnp.maximum(ap[t], bp[TOPK - 1 - t]) for t in range(TOPK)]
    for d in (8, 4, 2, 1):
        for t in range(TOPK):
            if t & d == 0:
                _ce(c, t, t + d)
    return c


def _topk_kernel(q_ref, k_ref, ov_ref, oi_ref, sc_ref):
    l_idx = pl.program_id(1)
    h = pl.program_id(2)
    neg_inf = jnp.float32(-jnp.inf)

    q_h = q_ref[0, h, :, :]
    k_h = k_ref[0, :, h, :]
    res = jax.lax.dot_general(
        k_h, q_h,
        dimension_numbers=(((1,), (0,)), ((), ())),
        preferred_element_type=jnp.float32,
    )
    for j in range(NC):
        sc_ref[:, j, :] = res[:, j * 128:(j + 1) * 128]

    tq = (l_idx * LT
          + 128 * jax.lax.broadcasted_iota(jnp.int32, (NC, 128), 0)
          + jax.lax.broadcasted_iota(jnp.int32, (NC, 128), 1))

    def sorted_group(g):
        p = []
        for t in range(TOPK):
            s = g * TOPK + t
            raw = jax.lax.bitcast_convert_type(sc_ref[s], jnp.int32)
            pk = jax.lax.bitcast_convert_type(
                (raw & jnp.int32(~127)) | (jnp.int32(127) - s), jnp.float32)
            p.append(jnp.where(tq >= BLOCK_SIZE * s + (BLOCK_SIZE + WINDOW - 1),
                               pk, neg_inf))
        for a, b in _PAIRS16:
            _ce(p, a, b)
        return p

    p0 = sorted_group(0)

    def body(g, carry):
        return tuple(_merge_top16(list(carry), sorted_group(g)))

    fp = list(jax.lax.fori_loop(1, l_idx + 1, body, tuple(p0)))

    for t in range(TOPK):
        pk = fp[t]
        is_ninf = pk == neg_inf
        bits = jax.lax.bitcast_convert_type(pk, jnp.int32)
        vt = jax.lax.bitcast_convert_type(bits & jnp.int32(~127), jnp.float32)
        vt = jnp.where(is_ninf, neg_inf, vt)
        it = jnp.where(is_ninf, jnp.int32(-1),
                       jnp.int32(127) - (bits & jnp.int32(127)))
        ov_ref[0, 0, h * TOPK + t] = vt
        oi_ref[0, 0, h * TOPK + t] = it


@jax.jit
def kernel(q, k):
    k_scaled = k * jnp.float32(SM_SCALE)
    q_t = q.transpose(0, 2, 3, 1)
    nl = L // LT
    grid = (B, nl, H)
    out_shape = (
        jax.ShapeDtypeStruct((B, nl, H * TOPK, NC, 128), jnp.float32),
        jax.ShapeDtypeStruct((B, nl, H * TOPK, NC, 128), jnp.int32),
    )
    q_spec = pl.BlockSpec((1, H, D, LT), lambda b, l, h: (b, 0, 0, l))
    k_spec = pl.BlockSpec((1, S, H, D), lambda b, l, h: (b, 0, 0, 0))
    o_spec = pl.BlockSpec((1, 1, H * TOPK, NC, 128), lambda b, l, h: (b, l, 0, 0, 0))
    ov, oi = pl.pallas_call(
        _topk_kernel,
        grid=grid,
        in_specs=[q_spec, k_spec],
        out_specs=(o_spec, o_spec),
        out_shape=out_shape,
        scratch_shapes=[pltpu.VMEM((S, NC, 128), jnp.float32)],
    )(q_t, k_scaled)
    ov = ov.transpose(0, 1, 3, 4, 2).reshape(B, L, H, TOPK)
    oi = oi.transpose(0, 1, 3, 4, 2).reshape(B, L, H, TOPK)
    return ov, oi
